# baseline (speedup 1.0000x reference)
import functools

import jax
import jax.numpy as jnp
from jax.experimental import pallas as pl
from jax.experimental.pallas import tpu as pltpu

R = 128
C = 100000
W = 640
NBLK = -(-C // W)

_ROT0 = (13, 15, 26, 6)
_ROT1 = (17, 29, 16, 24)


def _rotl(x, r):
    return (x << jnp.uint32(r)) | (x >> jnp.uint32(32 - r))


def _threefry_bits(n):
    ks0 = jnp.uint32(0)
    ks1 = jnp.uint32(42)
    ks2 = ks0 ^ ks1 ^ jnp.uint32(0x1BD11BDA)
    ks = (ks0, ks1, ks2)
    x0 = jnp.zeros_like(n) + ks0
    x1 = n + ks1
    for g in range(5):
        rots = _ROT0 if g % 2 == 0 else _ROT1
        for r in rots:
            x0 = x0 + x1
            x1 = _rotl(x1, r)
            x1 = x1 ^ x0
        x0 = x0 + ks[(g + 1) % 3]
        x1 = x1 + ks[(g + 2) % 3] + jnp.uint32(g + 1)
    return x0 ^ x1


def _body(x_ref, o_ref, val_ref, idx_ref):
    b = pl.program_id(0)

    @pl.when(b == 0)
    def _init():
        val_ref[...] = jnp.full((R, 1), -1.0, jnp.float32)
        idx_ref[...] = jnp.zeros((R, 1), jnp.int32)

    x = x_ref[...]
    m = jnp.max(x, axis=0, keepdims=True)
    e = jnp.exp(x - m)
    s = jnp.sum(e, axis=0, keepdims=True)

    row = jax.lax.broadcasted_iota(jnp.uint32, (R, W), 0)
    col = jax.lax.broadcasted_iota(jnp.uint32, (R, W), 1)
    gcol = jnp.uint32(W) * b.astype(jnp.uint32) + col
    n = row * jnp.uint32(C) + gcol
    bits = _threefry_bits(n)

    tiny = jnp.float32(jnp.finfo(jnp.float32).tiny)
    fb = (bits >> jnp.uint32(9)) | jnp.uint32(0x3F800000)
    f = jax.lax.bitcast_convert_type(fb, jnp.float32) - jnp.float32(1.0)
    u = jnp.maximum(f, tiny)
    t = -jnp.log(u)

    score = e / (s * t)
    score = jnp.where(gcol < jnp.uint32(C), score, jnp.float32(-1.0))

    bm = jnp.max(score, axis=1, keepdims=True)
    is_max = score == bm
    cand_idx = jnp.min(
        jnp.where(is_max, gcol.astype(jnp.int32), jnp.int32(0x7FFFFFFF)),
        axis=1, keepdims=True)

    better = bm > val_ref[...]
    val_ref[...] = jnp.where(better, bm, val_ref[...])
    idx_ref[...] = jnp.where(better, cand_idx, idx_ref[...])

    @pl.when(b == NBLK - 1)
    def _emit():
        o_ref[...] = idx_ref[...]


@jax.jit
def kernel(outputs):
    return pl.pallas_call(
        _body,
        grid=(NBLK,),
        in_specs=[pl.BlockSpec((R, W), lambda b: (0, b))],
        out_specs=pl.BlockSpec((R, 1), lambda b: (0, 0)),
        out_shape=jax.ShapeDtypeStruct((R, 1), jnp.int32),
        scratch_shapes=[
            pltpu.VMEM((R, 1), jnp.float32),
            pltpu.VMEM((R, 1), jnp.int32),
        ],
    )(outputs)


# speedup vs baseline: 1.0601x; 1.0601x over previous
import functools

import jax
import jax.numpy as jnp
from jax.experimental import pallas as pl
from jax.experimental.pallas import tpu as pltpu

R = 128
C = 100000
W = 1024
NBLK = -(-C // W)

_ROT0 = (13, 15, 26, 6)
_ROT1 = (17, 29, 16, 24)


def _rotl(x, r):
    return (x << jnp.uint32(r)) | (x >> jnp.uint32(32 - r))


def _threefry_bits(n):
    ks0 = jnp.uint32(0)
    ks1 = jnp.uint32(42)
    ks2 = ks0 ^ ks1 ^ jnp.uint32(0x1BD11BDA)
    ks = (ks0, ks1, ks2)
    x0 = jnp.zeros_like(n) + ks0
    x1 = n + ks1
    for g in range(5):
        rots = _ROT0 if g % 2 == 0 else _ROT1
        for r in rots:
            x0 = x0 + x1
            x1 = _rotl(x1, r)
            x1 = x1 ^ x0
        x0 = x0 + ks[(g + 1) % 3]
        x1 = x1 + ks[(g + 2) % 3] + jnp.uint32(g + 1)
    return x0 ^ x1


def _body(x_ref, o_ref, val_ref, idx_ref):
    b = pl.program_id(0)

    @pl.when(b == 0)
    def _init():
        val_ref[...] = jnp.full((R, 1), -1.0, jnp.float32)
        idx_ref[...] = jnp.zeros((R, 1), jnp.int32)

    x = x_ref[...]
    m = jnp.max(x, axis=0, keepdims=True)
    e = jnp.exp(x - m)
    s = jnp.sum(e, axis=0, keepdims=True)

    row = jax.lax.broadcasted_iota(jnp.uint32, (R, W), 0)
    col = jax.lax.broadcasted_iota(jnp.uint32, (R, W), 1)
    gcol = jnp.uint32(W) * b.astype(jnp.uint32) + col
    n = row * jnp.uint32(C) + gcol
    bits = _threefry_bits(n)

    tiny = jnp.float32(jnp.finfo(jnp.float32).tiny)
    fb = (bits >> jnp.uint32(9)) | jnp.uint32(0x3F800000)
    f = jax.lax.bitcast_convert_type(fb, jnp.float32) - jnp.float32(1.0)
    u = jnp.maximum(f, tiny)
    t = -jnp.log(u)

    score = e / (s * t)
    score = jnp.where(gcol < jnp.uint32(C), score, jnp.float32(-1.0))

    bm = jnp.max(score, axis=1, keepdims=True)
    is_max = score == bm
    cand_idx = jnp.min(
        jnp.where(is_max, gcol.astype(jnp.int32), jnp.int32(0x7FFFFFFF)),
        axis=1, keepdims=True)

    better = bm > val_ref[...]
    val_ref[...] = jnp.where(better, bm, val_ref[...])
    idx_ref[...] = jnp.where(better, cand_idx, idx_ref[...])

    @pl.when(b == NBLK - 1)
    def _emit():
        o_ref[...] = idx_ref[...]


@jax.jit
def kernel(outputs):
    return pl.pallas_call(
        _body,
        grid=(NBLK,),
        in_specs=[pl.BlockSpec((R, W), lambda b: (0, b))],
        out_specs=pl.BlockSpec((R, 1), lambda b: (0, 0)),
        out_shape=jax.ShapeDtypeStruct((R, 1), jnp.int32),
        scratch_shapes=[
            pltpu.VMEM((R, 1), jnp.float32),
            pltpu.VMEM((R, 1), jnp.int32),
        ],
    )(outputs)
